# Initial kernel scaffold; baseline (speedup 1.0000x reference)
import jax
import jax.numpy as jnp
from jax.experimental import pallas as pl


def kernel(x, edge_index, edge_type, batch, emb, W1, root1, b1, W2, root2, b2, Wl, bl):
    raise NotImplementedError("write your pallas kernel here")


# speedup vs baseline: 1.7334x; 1.7334x over previous
import functools
import jax
import jax.numpy as jnp
from jax.experimental import pallas as pl
from jax.experimental.pallas import tpu as pltpu

N = 50000
E = 800000
REL = 3
D = 64
B = 128
CLS = 2

BLK = 256
NP = ((N + BLK - 1) // BLK) * BLK


def _dense_layer_body(h_ref, s_ref, deg_ref, w_ref, root_ref, b_ref, o_ref):
    acc = jnp.dot(h_ref[...], root_ref[...], preferred_element_type=jnp.float32)
    acc = acc + b_ref[...]
    for r in range(REL):
        invd = 1.0 / jnp.clip(deg_ref[r], 1.0)
        acc = acc + jnp.dot(s_ref[r] * invd[:, None], w_ref[r],
                            preferred_element_type=jnp.float32)
    o_ref[...] = jnp.maximum(acc, 0.0)


def _dense_layer(h, s, deg, w, root, b):
    grid = NP // BLK
    return pl.pallas_call(
        _dense_layer_body,
        grid=(grid,),
        in_specs=[
            pl.BlockSpec((BLK, D), lambda i: (i, 0)),
            pl.BlockSpec((REL, BLK, D), lambda i: (0, i, 0)),
            pl.BlockSpec((REL, BLK), lambda i: (0, i)),
            pl.BlockSpec((REL, D, D), lambda i: (0, 0, 0)),
            pl.BlockSpec((D, D), lambda i: (0, 0)),
            pl.BlockSpec((1, D), lambda i: (0, 0)),
        ],
        out_specs=pl.BlockSpec((BLK, D), lambda i: (i, 0)),
        out_shape=jax.ShapeDtypeStruct((NP, D), jnp.float32),
    )(h, s, deg, w, root, b)


def _head_body(sums_ref, cnt_ref, wl_ref, bl_ref, o_ref):
    pooled = sums_ref[...] / jnp.clip(cnt_ref[...], 1.0)
    o_ref[...] = jnp.dot(pooled, wl_ref[...],
                         preferred_element_type=jnp.float32) + bl_ref[...]


def _head(sums, cnt, wl, bl):
    return pl.pallas_call(
        _head_body,
        out_shape=jax.ShapeDtypeStruct((B, CLS), jnp.float32),
    )(sums, cnt, wl, bl)


def kernel(x, edge_index, edge_type, batch, emb, W1, root1, b1, W2, root2, b2, Wl, bl):
    emb0 = emb.at[0].set(0.0)
    h = emb0[x]
    h = jnp.pad(h, ((0, NP - N), (0, 0)))
    src, dst = edge_index[0], edge_index[1]
    ids = edge_type * NP + dst
    ones = jnp.ones((E,), jnp.float32)

    def layer(h, w, root, b):
        s = jax.ops.segment_sum(h[src], ids, num_segments=REL * NP)
        s = s.reshape(REL, NP, D)
        deg = jax.ops.segment_sum(ones, ids, num_segments=REL * NP).reshape(REL, NP)
        return _dense_layer(h, s, deg, w, root, b.reshape(1, D))

    h = layer(h, W1, root1, b1)
    h = layer(h, W2, root2, b2)
    h = h[:N]

    sums = jax.ops.segment_sum(h, batch, num_segments=B)
    cnt = jax.ops.segment_sum(jnp.ones((N,), jnp.float32), batch, num_segments=B)
    return _head(sums, cnt.reshape(B, 1), Wl, bl.reshape(1, CLS))


# speedup vs baseline: 2.1635x; 1.2481x over previous
import functools
import jax
import jax.numpy as jnp
from jax import lax
from jax.experimental import pallas as pl
from jax.experimental.pallas import tpu as pltpu
from jax.experimental.pallas import tpu_sc as plsc

N = 50000
E = 800000
REL = 3
D = 64
B = 128
CLS = 2

BLK = 256
NP = 53248

NC = 6400
NRANGE_PER_CORE = 4
TRASH = REL * NC
SLAB = 19328
CH = 128
NBUF = 3
EPT = 50000
NFULL = 390
_PZ = True
_PT = True
_PW = True
_PD = True
_PG = True
TAIL = 80

_mesh = plsc.VectorSubcoreMesh(core_axis_name="c", subcore_axis_name="s")
_sc_params = pltpu.CompilerParams(use_tc_tiling_on_sc=False)


@functools.partial(
    pl.kernel,
    out_type=jax.ShapeDtypeStruct((NP, D), jnp.float32),
    mesh=_mesh,
    scratch_types=[
        pltpu.VMEM((1664,), jnp.int32),
        pltpu.VMEM((1664, D), jnp.float32),
        pltpu.SemaphoreType.DMA,
    ],
    compiler_params=_sc_params,
)
def _sc_embed(x_hbm, emb_hbm, out_hbm, idx_v, rows_v, sem):
    s = lax.axis_index("s")
    c = lax.axis_index("c")
    wid = s * 2 + c
    base = wid * 1664
    pltpu.sync_copy(x_hbm.at[pl.ds(base, 1664)], idx_v)
    descs = []
    for k in range(13):
        descs.append(pltpu.async_copy(
            emb_hbm.at[idx_v.at[pl.ds(k * 128, 128)]],
            rows_v.at[pl.ds(k * 128, 128)], sem))
    for d_ in descs:
        d_.wait()
    pltpu.sync_copy(rows_v, out_hbm.at[pl.ds(base, 1664)])


def _make_sc_agg(with_deg: bool):
    out_type = [jax.ShapeDtypeStruct((REL, NP, D), jnp.float32)]
    if with_deg:
        out_type.append(jax.ShapeDtypeStruct((REL, NP, 8), jnp.float32))
    scratch = [
        [pltpu.VMEM((3, CH), jnp.int32) for _ in range(NBUF)],
        [pltpu.VMEM((CH, D), jnp.float32) for _ in range(NBUF)],
        [pltpu.VMEM((CH,), jnp.int32) for _ in range(NBUF)],
        pltpu.VMEM((TAIL,), jnp.int32),
        pltpu.VMEM((CH, 8), jnp.float32),
        pltpu.VMEM((CH, 8), jnp.float32),
        [pltpu.SemaphoreType.DMA for _ in range(NBUF)],
        [pltpu.SemaphoreType.DMA for _ in range(NBUF)],
        pltpu.VMEM_SHARED((SLAB, D), jnp.float32),
        pltpu.VMEM_SHARED((SLAB, 8), jnp.float32),
    ]

    def compute_rows(ebuf, idxbuf, base, n16):
        for j in range(n16):
            d_ = ebuf[1, pl.ds(j * 16, 16)]
            t_ = ebuf[2, pl.ds(j * 16, 16)]
            inb = (d_ >= base) & (d_ < base + NC)
            row = jnp.where(inb, t_ * NC + (d_ - base), TRASH + (d_ & 127))
            idxbuf[pl.ds(j * 16, 16)] = row

    def body(h_hbm, edges_hbm, ones_hbm, zrow_hbm, zdeg_hbm, *rest):
        if with_deg:
            s_out, deg_out = rest[0], rest[1]
            rest = rest[2:]
        else:
            s_out = rest[0]
            rest = rest[1:]
        (ebufs, rowbufs, idxbufs, idxtail, onesb, zdbuf, esems, gsems,
         slab, slab_deg) = rest

        s = lax.axis_index("s")
        c = lax.axis_index("c")
        pltpu.sync_copy(ones_hbm, onesb)
        e0 = s * EPT

        for p in range(NRANGE_PER_CORE):
            r = c * NRANGE_PER_CORE + p
            base = r * NC
            if _PZ:
                pltpu.sync_copy(zrow_hbm.at[pl.ds(0, CH)], rowbufs[0])
                pltpu.sync_copy(zdeg_hbm.at[pl.ds(0, CH)], zdbuf)
                iota16 = lax.iota(jnp.int32, 16)
                for z in range(10):
                    z0 = s * 1208 + (z * 128 if z < 9 else 1080)
                    for j in range(8):
                        idxbufs[0][pl.ds(j * 16, 16)] = z0 + j * 16 + iota16
                    pltpu.sync_copy(rowbufs[0], slab.at[idxbufs[0]])
                    if with_deg and _PD:
                        pltpu.sync_copy(zdbuf, slab_deg.at[idxbufs[0]])
            plsc.subcore_barrier()

            def group(g, _):
                for b_ in range(NBUF):
                    off = e0 + (g * NBUF + b_) * CH
                    pltpu.async_copy(
                        edges_hbm.at[:, pl.ds(off, CH)], ebufs[b_],
                        esems[b_]).wait()
                    pltpu.async_copy(
                        h_hbm.at[ebufs[b_].at[0]], rowbufs[b_],
                        gsems[b_]).wait()
                    compute_rows(ebufs[b_], idxbufs[b_], base, CH // 16)
                    pltpu.sync_copy(rowbufs[b_], slab.at[idxbufs[b_]], add=True)
                    if with_deg:
                        pltpu.sync_copy(onesb, slab_deg.at[idxbufs[b_]],
                                        add=True)
                return 0

            if _PG:
                lax.fori_loop(0, NFULL // NBUF, group, 0)

            if _PT:
                toff = e0 + NFULL * CH
                pltpu.sync_copy(edges_hbm.at[:, pl.ds(toff, TAIL)],
                                ebufs[0].at[:, pl.ds(0, TAIL)])
                pltpu.async_copy(h_hbm.at[ebufs[0].at[0, pl.ds(0, TAIL)]],
                                 rowbufs[0].at[pl.ds(0, TAIL)],
                                 gsems[0]).wait()
                for j in range(TAIL // 16):
                    d_ = ebufs[0][1, pl.ds(j * 16, 16)]
                    t_ = ebufs[0][2, pl.ds(j * 16, 16)]
                    inb = (d_ >= base) & (d_ < base + NC)
                    row = jnp.where(inb, t_ * NC + (d_ - base),
                                    TRASH + (d_ & 127))
                    idxtail[pl.ds(j * 16, 16)] = row
                pltpu.sync_copy(rowbufs[0].at[pl.ds(0, TAIL)],
                                slab.at[idxtail], add=True)
                if with_deg:
                    pltpu.sync_copy(onesb.at[pl.ds(0, TAIL)],
                                    slab_deg.at[idxtail], add=True)

            plsc.subcore_barrier()
            if _PW:
                iota16 = lax.iota(jnp.int32, 16)

                def stage_out(starts):
                    for rr in range(REL):
                        for po in starts:
                            src0 = rr * NC + s * 400 + po
                            dst0 = base + s * 400 + po
                            for j in range(8):
                                idxbufs[0][pl.ds(j * 16, 16)] = (
                                    src0 + j * 16 + iota16)
                            pltpu.sync_copy(slab.at[idxbufs[0]], rowbufs[1])
                            pltpu.sync_copy(rowbufs[1],
                                            s_out.at[rr, pl.ds(dst0, CH)])
                            if with_deg:
                                pltpu.sync_copy(slab_deg.at[idxbufs[0]], zdbuf)
                                pltpu.sync_copy(
                                    zdbuf, deg_out.at[rr, pl.ds(dst0, CH)])

                stage_out([0, 128, 256, 272])
            plsc.subcore_barrier()

    return pl.kernel(body, out_type=tuple(out_type) if with_deg else out_type[0],
                     mesh=_mesh, scratch_types=scratch,
                     compiler_params=_sc_params)


_sc_agg_deg = _make_sc_agg(True)


@functools.partial(
    pl.kernel,
    out_type=(jax.ShapeDtypeStruct((2, 256, D), jnp.float32),
              jax.ShapeDtypeStruct((2, 256, 8), jnp.float32)),
    mesh=_mesh,
    scratch_types=[
        pltpu.VMEM((128, D), jnp.float32),
        pltpu.VMEM((128,), jnp.int32),
        pltpu.VMEM((128, 8), jnp.float32),
        pltpu.SemaphoreType.DMA,
        pltpu.VMEM_SHARED((256, D), jnp.float32),
        pltpu.VMEM_SHARED((256, 8), jnp.float32),
    ],
    compiler_params=_sc_params,
)
def _sc_pool(h_hbm, b_hbm, ones_hbm, zrow_hbm, zdeg_hbm, psum_out, pcnt_out,
             rows_v, bidx_v, onesb, sem, slab, slab_cnt):
    s = lax.axis_index("s")
    c = lax.axis_index("c")
    wid = s * 2 + c
    pltpu.sync_copy(ones_hbm.at[pl.ds(0, 128)], onesb)

    pltpu.sync_copy(zrow_hbm.at[pl.ds(0, 16)], slab.at[pl.ds(s * 16, 16)])
    pltpu.sync_copy(zdeg_hbm.at[pl.ds(0, 16)],
                    slab_cnt.at[pl.ds(s * 16, 16)])
    plsc.subcore_barrier()
    for k in range(13):
        ch = wid * 13 + k
        pltpu.sync_copy(h_hbm.at[pl.ds(ch * 128, 128)], rows_v)
        pltpu.sync_copy(b_hbm.at[pl.ds(ch * 128, 128)], bidx_v)
        pltpu.sync_copy(rows_v, slab.at[bidx_v], add=True)
        pltpu.sync_copy(onesb, slab_cnt.at[bidx_v], add=True)
    plsc.subcore_barrier()
    pltpu.sync_copy(slab.at[pl.ds(s * 16, 16)],
                    psum_out.at[c, pl.ds(s * 16, 16)])
    pltpu.sync_copy(slab_cnt.at[pl.ds(s * 16, 16)],
                    pcnt_out.at[c, pl.ds(s * 16, 16)])


def _dense_layer_body(h_ref, s_ref, deg_ref, w_ref, root_ref, b_ref, o_ref):
    acc = jnp.dot(h_ref[...], root_ref[...], preferred_element_type=jnp.float32)
    acc = acc + b_ref[...]
    for r in range(REL):
        invd = 1.0 / jnp.clip(deg_ref[r, :, 0], 1.0)
        acc = acc + jnp.dot(s_ref[r] * invd[:, None], w_ref[r],
                            preferred_element_type=jnp.float32)
    o_ref[...] = jnp.maximum(acc, 0.0)


def _dense_layer(h, s, deg, w, root, b):
    grid = NP // BLK
    return pl.pallas_call(
        _dense_layer_body,
        grid=(grid,),
        in_specs=[
            pl.BlockSpec((BLK, D), lambda i: (i, 0)),
            pl.BlockSpec((REL, BLK, D), lambda i: (0, i, 0)),
            pl.BlockSpec((REL, BLK, 8), lambda i: (0, i, 0)),
            pl.BlockSpec((REL, D, D), lambda i: (0, 0, 0)),
            pl.BlockSpec((D, D), lambda i: (0, 0)),
            pl.BlockSpec((1, D), lambda i: (0, 0)),
        ],
        out_specs=pl.BlockSpec((BLK, D), lambda i: (i, 0)),
        out_shape=jax.ShapeDtypeStruct((NP, D), jnp.float32),
    )(h, s, deg, w, root, b)


def _head_body(psum_ref, pcnt_ref, wl_ref, bl_ref, o_ref):
    sums = psum_ref[0, :B, :] + psum_ref[1, :B, :]
    cnt = pcnt_ref[0, :B, 0:1] + pcnt_ref[1, :B, 0:1]
    pooled = sums / jnp.clip(cnt, 1.0)
    o_ref[...] = jnp.dot(pooled, wl_ref[...],
                         preferred_element_type=jnp.float32) + bl_ref[...]


def _head(psum, pcnt, wl, bl):
    return pl.pallas_call(
        _head_body,
        out_shape=jax.ShapeDtypeStruct((B, CLS), jnp.float32),
    )(psum, pcnt, wl, bl)


def kernel(x, edge_index, edge_type, batch, emb, W1, root1, b1, W2, root2, b2, Wl, bl):
    emb0 = emb.at[0].set(0.0)
    x_pad = jnp.concatenate([x, jnp.zeros((NP - N,), jnp.int32)])
    b_pad = jnp.concatenate([batch, jnp.full((NP - N,), B, jnp.int32)])
    edges3 = jnp.concatenate([edge_index, edge_type[None]], axis=0)

    ones_c = jnp.ones((CH, 8), jnp.float32)
    zrow_c = jnp.zeros((400, D), jnp.float32)
    zdeg_c = jnp.zeros((CH, 8), jnp.float32)

    _STAGE = 3
    src, dst, ty = edges3[0], edges3[1], edges3[2]
    h0 = _sc_embed(x_pad, emb0)

    def _xla_agg(h):
        ids = ty * NP + dst
        s_ = jax.ops.segment_sum(h[src], ids, num_segments=REL * NP)
        d_ = jax.ops.segment_sum(jnp.ones((E,), jnp.float32), ids,
                                 num_segments=REL * NP)
        return s_.reshape(REL, NP, D), d_.reshape(REL, NP, 1)

    if _STAGE >= 2:
        s1, deg = _sc_agg_deg(h0, edges3, ones_c, zrow_c, zdeg_c)
    else:
        s1, deg = _xla_agg(h0)
    h1 = _dense_layer(h0, s1, deg, W1, root1, b1.reshape(1, D))
    if _STAGE >= 2:
        s2, _deg2 = _sc_agg_deg(h1, edges3, ones_c, zrow_c, zdeg_c)
    else:
        s2, _deg2 = _xla_agg(h1)
    h2 = _dense_layer(h1, s2, deg, W2, root2, b2.reshape(1, D))
    if _STAGE >= 3:
        psum, pcnt = _sc_pool(h2, b_pad, ones_c, zrow_c, zdeg_c)
    else:
        hs = h2[:N]
        sums = jax.ops.segment_sum(hs, batch, num_segments=B)
        cnt = jax.ops.segment_sum(jnp.ones((N,), jnp.float32), batch,
                                  num_segments=B)
        psum = jnp.zeros((2, 256, D), jnp.float32).at[0, :B].set(sums)
        pcnt = jnp.zeros((2, 256, 1), jnp.float32).at[0, :B].set(cnt[:, None])
    return _head(psum, pcnt, Wl, bl.reshape(1, CLS))


# speedup vs baseline: 3.5546x; 1.6430x over previous
import functools
import jax
import jax.numpy as jnp
from jax import lax
from jax.experimental import pallas as pl
from jax.experimental.pallas import tpu as pltpu
from jax.experimental.pallas import tpu_sc as plsc

N = 50000
E = 800000
REL = 3
D = 64
B = 128
CLS = 2

BLK = 256
NP = 53248

NC = 6400
NRANGE_PER_CORE = 4
TRASH = REL * NC
SLAB = 19328
CH = 128
NBUF = 3
EPT = 50000
NFULL = 390
_PZ = True
_PT = True
_PW = True
_PD = True
_PG = True
TAIL = 80

_mesh = plsc.VectorSubcoreMesh(core_axis_name="c", subcore_axis_name="s")
_sc_params = pltpu.CompilerParams(use_tc_tiling_on_sc=False)


@functools.partial(
    pl.kernel,
    out_type=jax.ShapeDtypeStruct((NP, D), jnp.float32),
    mesh=_mesh,
    scratch_types=[
        pltpu.VMEM((1664,), jnp.int32),
        pltpu.VMEM((1664, D), jnp.float32),
        pltpu.SemaphoreType.DMA,
    ],
    compiler_params=_sc_params,
)
def _sc_embed(x_hbm, emb_hbm, out_hbm, idx_v, rows_v, sem):
    s = lax.axis_index("s")
    c = lax.axis_index("c")
    wid = s * 2 + c
    base = wid * 1664
    pltpu.sync_copy(x_hbm.at[pl.ds(base, 1664)], idx_v)
    descs = []
    for k in range(13):
        descs.append(pltpu.async_copy(
            emb_hbm.at[idx_v.at[pl.ds(k * 128, 128)]],
            rows_v.at[pl.ds(k * 128, 128)], sem))
    for d_ in descs:
        d_.wait()
    pltpu.sync_copy(rows_v, out_hbm.at[pl.ds(base, 1664)])


def _make_sc_agg(with_deg: bool):
    out_type = [jax.ShapeDtypeStruct((REL, NP, D), jnp.float32)]
    if with_deg:
        out_type.append(jax.ShapeDtypeStruct((REL, NP, 8), jnp.float32))
    scratch = [
        [pltpu.VMEM((3, CH), jnp.int32) for _ in range(NBUF)],
        [pltpu.VMEM((CH, D), jnp.float32) for _ in range(NBUF)],
        [pltpu.VMEM((CH,), jnp.int32) for _ in range(NBUF)],
        pltpu.VMEM((TAIL,), jnp.int32),
        pltpu.VMEM((CH, 8), jnp.float32),
        pltpu.VMEM((CH, 8), jnp.float32),
        [pltpu.SemaphoreType.DMA for _ in range(NBUF)],
        [pltpu.SemaphoreType.DMA for _ in range(NBUF)],
        [pltpu.SemaphoreType.DMA for _ in range(NBUF)],
        [pltpu.SemaphoreType.DMA for _ in range(NBUF)],
        pltpu.VMEM_SHARED((SLAB, D), jnp.float32),
        pltpu.VMEM_SHARED((SLAB, 8), jnp.float32),
    ]

    def compute_rows(ebuf, idxbuf, base, n16):
        for j in range(n16):
            d_ = ebuf[1, pl.ds(j * 16, 16)]
            t_ = ebuf[2, pl.ds(j * 16, 16)]
            inb = (d_ >= base) & (d_ < base + NC)
            row = jnp.where(inb, t_ * NC + (d_ - base), TRASH + (d_ & 127))
            idxbuf[pl.ds(j * 16, 16)] = row

    def body(h_hbm, edges_hbm, ones_hbm, zrow_hbm, zdeg_hbm, *rest):
        if with_deg:
            s_out, deg_out = rest[0], rest[1]
            rest = rest[2:]
        else:
            s_out = rest[0]
            rest = rest[1:]
        (ebufs, rowbufs, idxbufs, idxtail, onesb, zdbuf, esems, gsems,
         ssems, dsems, slab, slab_deg) = rest

        s = lax.axis_index("s")
        c = lax.axis_index("c")
        pltpu.sync_copy(ones_hbm, onesb)
        e0 = s * EPT

        for p in range(NRANGE_PER_CORE):
            r = c * NRANGE_PER_CORE + p
            base = r * NC
            if _PZ:
                pltpu.sync_copy(zrow_hbm.at[pl.ds(0, CH)], rowbufs[0])
                pltpu.sync_copy(zdeg_hbm.at[pl.ds(0, CH)], zdbuf)
                iota16 = lax.iota(jnp.int32, 16)
                for z in range(10):
                    z0 = s * 1208 + (z * 128 if z < 9 else 1080)
                    for j in range(8):
                        idxbufs[0][pl.ds(j * 16, 16)] = z0 + j * 16 + iota16
                    pltpu.sync_copy(rowbufs[0], slab.at[idxbufs[0]])
                    if with_deg and _PD:
                        pltpu.sync_copy(zdbuf, slab_deg.at[idxbufs[0]])
            plsc.subcore_barrier()

            def group(g, _):
                edescs = []
                for b_ in range(NBUF):
                    off = e0 + (g * NBUF + b_) * CH
                    edescs.append(pltpu.async_copy(
                        edges_hbm.at[:, pl.ds(off, CH)], ebufs[b_],
                        esems[b_]))
                gdescs = []
                for b_ in range(NBUF):
                    edescs[b_].wait()
                    gdescs.append(pltpu.async_copy(
                        h_hbm.at[ebufs[b_].at[0]], rowbufs[b_], gsems[b_]))
                sdescs = []
                for b_ in range(NBUF):
                    compute_rows(ebufs[b_], idxbufs[b_], base, CH // 16)
                    gdescs[b_].wait()
                    sdescs.append(pltpu.async_copy(
                        rowbufs[b_], slab.at[idxbufs[b_]], ssems[b_],
                        add=True))
                    if with_deg:
                        sdescs.append(pltpu.async_copy(
                            onesb, slab_deg.at[idxbufs[b_]], dsems[b_],
                            add=True))
                for d_ in sdescs:
                    d_.wait()
                return 0

            if _PG:
                lax.fori_loop(0, NFULL // NBUF, group, 0)

            if _PT:
                toff = e0 + NFULL * CH
                pltpu.sync_copy(edges_hbm.at[:, pl.ds(toff, TAIL)],
                                ebufs[0].at[:, pl.ds(0, TAIL)])
                pltpu.async_copy(h_hbm.at[ebufs[0].at[0, pl.ds(0, TAIL)]],
                                 rowbufs[0].at[pl.ds(0, TAIL)],
                                 gsems[0]).wait()
                for j in range(TAIL // 16):
                    d_ = ebufs[0][1, pl.ds(j * 16, 16)]
                    t_ = ebufs[0][2, pl.ds(j * 16, 16)]
                    inb = (d_ >= base) & (d_ < base + NC)
                    row = jnp.where(inb, t_ * NC + (d_ - base),
                                    TRASH + (d_ & 127))
                    idxtail[pl.ds(j * 16, 16)] = row
                pltpu.sync_copy(rowbufs[0].at[pl.ds(0, TAIL)],
                                slab.at[idxtail], add=True)
                if with_deg:
                    pltpu.sync_copy(onesb.at[pl.ds(0, TAIL)],
                                    slab_deg.at[idxtail], add=True)

            plsc.subcore_barrier()
            if _PW:
                iota16 = lax.iota(jnp.int32, 16)

                def stage_out(starts):
                    for rr in range(REL):
                        for po in starts:
                            src0 = rr * NC + s * 400 + po
                            dst0 = base + s * 400 + po
                            for j in range(8):
                                idxbufs[0][pl.ds(j * 16, 16)] = (
                                    src0 + j * 16 + iota16)
                            pltpu.sync_copy(slab.at[idxbufs[0]], rowbufs[1])
                            pltpu.sync_copy(rowbufs[1],
                                            s_out.at[rr, pl.ds(dst0, CH)])
                            if with_deg:
                                pltpu.sync_copy(slab_deg.at[idxbufs[0]], zdbuf)
                                pltpu.sync_copy(
                                    zdbuf, deg_out.at[rr, pl.ds(dst0, CH)])

                stage_out([0, 128, 256, 272])
            plsc.subcore_barrier()

    return pl.kernel(body, out_type=tuple(out_type) if with_deg else out_type[0],
                     mesh=_mesh, scratch_types=scratch,
                     compiler_params=_sc_params)


_sc_agg_deg = _make_sc_agg(True)


@functools.partial(
    pl.kernel,
    out_type=(jax.ShapeDtypeStruct((2, 256, D), jnp.float32),
              jax.ShapeDtypeStruct((2, 256, 8), jnp.float32)),
    mesh=_mesh,
    scratch_types=[
        pltpu.VMEM((128, D), jnp.float32),
        pltpu.VMEM((128,), jnp.int32),
        pltpu.VMEM((128, 8), jnp.float32),
        pltpu.SemaphoreType.DMA,
        pltpu.VMEM_SHARED((256, D), jnp.float32),
        pltpu.VMEM_SHARED((256, 8), jnp.float32),
    ],
    compiler_params=_sc_params,
)
def _sc_pool(h_hbm, b_hbm, ones_hbm, zrow_hbm, zdeg_hbm, psum_out, pcnt_out,
             rows_v, bidx_v, onesb, sem, slab, slab_cnt):
    s = lax.axis_index("s")
    c = lax.axis_index("c")
    wid = s * 2 + c
    pltpu.sync_copy(ones_hbm.at[pl.ds(0, 128)], onesb)

    pltpu.sync_copy(zrow_hbm.at[pl.ds(0, 16)], slab.at[pl.ds(s * 16, 16)])
    pltpu.sync_copy(zdeg_hbm.at[pl.ds(0, 16)],
                    slab_cnt.at[pl.ds(s * 16, 16)])
    plsc.subcore_barrier()
    for k in range(13):
        ch = wid * 13 + k
        pltpu.sync_copy(h_hbm.at[pl.ds(ch * 128, 128)], rows_v)
        pltpu.sync_copy(b_hbm.at[pl.ds(ch * 128, 128)], bidx_v)
        pltpu.sync_copy(rows_v, slab.at[bidx_v], add=True)
        pltpu.sync_copy(onesb, slab_cnt.at[bidx_v], add=True)
    plsc.subcore_barrier()
    pltpu.sync_copy(slab.at[pl.ds(s * 16, 16)],
                    psum_out.at[c, pl.ds(s * 16, 16)])
    pltpu.sync_copy(slab_cnt.at[pl.ds(s * 16, 16)],
                    pcnt_out.at[c, pl.ds(s * 16, 16)])


def _dense_layer_body(h_ref, s_ref, deg_ref, w_ref, root_ref, b_ref, o_ref):
    acc = jnp.dot(h_ref[...], root_ref[...], preferred_element_type=jnp.float32)
    acc = acc + b_ref[...]
    for r in range(REL):
        invd = 1.0 / jnp.clip(deg_ref[r, :, 0], 1.0)
        acc = acc + jnp.dot(s_ref[r] * invd[:, None], w_ref[r],
                            preferred_element_type=jnp.float32)
    o_ref[...] = jnp.maximum(acc, 0.0)


def _dense_layer(h, s, deg, w, root, b):
    grid = NP // BLK
    return pl.pallas_call(
        _dense_layer_body,
        grid=(grid,),
        in_specs=[
            pl.BlockSpec((BLK, D), lambda i: (i, 0)),
            pl.BlockSpec((REL, BLK, D), lambda i: (0, i, 0)),
            pl.BlockSpec((REL, BLK, 8), lambda i: (0, i, 0)),
            pl.BlockSpec((REL, D, D), lambda i: (0, 0, 0)),
            pl.BlockSpec((D, D), lambda i: (0, 0)),
            pl.BlockSpec((1, D), lambda i: (0, 0)),
        ],
        out_specs=pl.BlockSpec((BLK, D), lambda i: (i, 0)),
        out_shape=jax.ShapeDtypeStruct((NP, D), jnp.float32),
    )(h, s, deg, w, root, b)


def _head_body(psum_ref, pcnt_ref, wl_ref, bl_ref, o_ref):
    sums = psum_ref[0, :B, :] + psum_ref[1, :B, :]
    cnt = pcnt_ref[0, :B, 0:1] + pcnt_ref[1, :B, 0:1]
    pooled = sums / jnp.clip(cnt, 1.0)
    o_ref[...] = jnp.dot(pooled, wl_ref[...],
                         preferred_element_type=jnp.float32) + bl_ref[...]


def _head(psum, pcnt, wl, bl):
    return pl.pallas_call(
        _head_body,
        out_shape=jax.ShapeDtypeStruct((B, CLS), jnp.float32),
    )(psum, pcnt, wl, bl)


def kernel(x, edge_index, edge_type, batch, emb, W1, root1, b1, W2, root2, b2, Wl, bl):
    emb0 = emb.at[0].set(0.0)
    x_pad = jnp.concatenate([x, jnp.zeros((NP - N,), jnp.int32)])
    b_pad = jnp.concatenate([batch, jnp.full((NP - N,), B, jnp.int32)])
    edges3 = jnp.concatenate([edge_index, edge_type[None]], axis=0)

    ones_c = jnp.ones((CH, 8), jnp.float32)
    zrow_c = jnp.zeros((400, D), jnp.float32)
    zdeg_c = jnp.zeros((CH, 8), jnp.float32)

    _STAGE = 3
    src, dst, ty = edges3[0], edges3[1], edges3[2]
    h0 = _sc_embed(x_pad, emb0)

    def _xla_agg(h):
        ids = ty * NP + dst
        s_ = jax.ops.segment_sum(h[src], ids, num_segments=REL * NP)
        d_ = jax.ops.segment_sum(jnp.ones((E,), jnp.float32), ids,
                                 num_segments=REL * NP)
        return s_.reshape(REL, NP, D), d_.reshape(REL, NP, 1)

    if _STAGE >= 2:
        s1, deg = _sc_agg_deg(h0, edges3, ones_c, zrow_c, zdeg_c)
    else:
        s1, deg = _xla_agg(h0)
    h1 = _dense_layer(h0, s1, deg, W1, root1, b1.reshape(1, D))
    if _STAGE >= 2:
        s2, _deg2 = _sc_agg_deg(h1, edges3, ones_c, zrow_c, zdeg_c)
    else:
        s2, _deg2 = _xla_agg(h1)
    h2 = _dense_layer(h1, s2, deg, W2, root2, b2.reshape(1, D))
    if _STAGE >= 3:
        psum, pcnt = _sc_pool(h2, b_pad, ones_c, zrow_c, zdeg_c)
    else:
        hs = h2[:N]
        sums = jax.ops.segment_sum(hs, batch, num_segments=B)
        cnt = jax.ops.segment_sum(jnp.ones((N,), jnp.float32), batch,
                                  num_segments=B)
        psum = jnp.zeros((2, 256, D), jnp.float32).at[0, :B].set(sums)
        pcnt = jnp.zeros((2, 256, 1), jnp.float32).at[0, :B].set(cnt[:, None])
    return _head(psum, pcnt, Wl, bl.reshape(1, CLS))


# speedup vs baseline: 3.6122x; 1.0162x over previous
import functools
import jax
import jax.numpy as jnp
from jax import lax
from jax.experimental import pallas as pl
from jax.experimental.pallas import tpu as pltpu
from jax.experimental.pallas import tpu_sc as plsc

N = 50000
E = 800000
REL = 3
D = 64
B = 128
CLS = 2

BLK = 256
NP = 53248

NC = 6400
NRANGE_PER_CORE = 4
TRASH = REL * NC
SLAB = 19328
CH = 128
NBUF = 3
EPT = 50000
NFULL = 390
_PZ = True
_PT = True
_PW = True
_PD = True
_PG = True
TAIL = 80

_mesh = plsc.VectorSubcoreMesh(core_axis_name="c", subcore_axis_name="s")
_sc_params = pltpu.CompilerParams(use_tc_tiling_on_sc=False)


@functools.partial(
    pl.kernel,
    out_type=jax.ShapeDtypeStruct((NP, D), jnp.float32),
    mesh=_mesh,
    scratch_types=[
        pltpu.VMEM((1664,), jnp.int32),
        pltpu.VMEM((1664, D), jnp.float32),
        pltpu.SemaphoreType.DMA,
    ],
    compiler_params=_sc_params,
)
def _sc_embed(x_hbm, emb_hbm, out_hbm, idx_v, rows_v, sem):
    s = lax.axis_index("s")
    c = lax.axis_index("c")
    wid = s * 2 + c
    base = wid * 1664
    pltpu.sync_copy(x_hbm.at[pl.ds(base, 1664)], idx_v)
    descs = []
    for k in range(13):
        descs.append(pltpu.async_copy(
            emb_hbm.at[idx_v.at[pl.ds(k * 128, 128)]],
            rows_v.at[pl.ds(k * 128, 128)], sem))
    for d_ in descs:
        d_.wait()
    pltpu.sync_copy(rows_v, out_hbm.at[pl.ds(base, 1664)])


def _make_sc_agg(with_deg: bool):
    out_type = [jax.ShapeDtypeStruct((REL, NP, D), jnp.float32)]
    if with_deg:
        out_type.append(jax.ShapeDtypeStruct((REL, NP, 8), jnp.float32))
    scratch = [
        [pltpu.VMEM((3, CH), jnp.int32) for _ in range(NBUF)],
        [pltpu.VMEM((CH, D), jnp.float32) for _ in range(NBUF)],
        [pltpu.VMEM((CH,), jnp.int32) for _ in range(NBUF)],
        pltpu.VMEM((TAIL,), jnp.int32),
        pltpu.VMEM((CH, 8), jnp.float32),
        pltpu.VMEM((CH, 8), jnp.float32),
        [pltpu.SemaphoreType.DMA for _ in range(NBUF)],
        [pltpu.SemaphoreType.DMA for _ in range(NBUF)],
        [pltpu.SemaphoreType.DMA for _ in range(NBUF)],
        [pltpu.SemaphoreType.DMA for _ in range(NBUF)],
        pltpu.VMEM_SHARED((SLAB, D), jnp.float32),
        pltpu.VMEM_SHARED((SLAB, 8), jnp.float32),
    ]

    def compute_rows(ebuf, idxbuf, base, n16):
        for j in range(n16):
            d_ = ebuf[1, pl.ds(j * 16, 16)]
            t_ = ebuf[2, pl.ds(j * 16, 16)]
            inb = (d_ >= base) & (d_ < base + NC)
            row = jnp.where(inb, t_ * NC + (d_ - base), TRASH + (d_ & 127))
            idxbuf[pl.ds(j * 16, 16)] = row

    def body(h_hbm, edges_hbm, ones_hbm, zrow_hbm, zdeg_hbm, *rest):
        if with_deg:
            s_out, deg_out = rest[0], rest[1]
            rest = rest[2:]
        else:
            s_out = rest[0]
            rest = rest[1:]
        (ebufs, rowbufs, idxbufs, idxtail, onesb, zdbuf, esems, gsems,
         ssems, dsems, slab, slab_deg) = rest

        s = lax.axis_index("s")
        c = lax.axis_index("c")
        pltpu.sync_copy(ones_hbm, onesb)
        e0 = s * EPT

        for p in range(NRANGE_PER_CORE):
            r = c * NRANGE_PER_CORE + p
            base = r * NC
            if _PZ:
                pltpu.sync_copy(zrow_hbm.at[pl.ds(0, CH)], rowbufs[0])
                pltpu.sync_copy(zdeg_hbm.at[pl.ds(0, CH)], zdbuf)
                iota16 = lax.iota(jnp.int32, 16)
                for z in range(10):
                    z0 = s * 1208 + (z * 128 if z < 9 else 1080)
                    for j in range(8):
                        idxbufs[0][pl.ds(j * 16, 16)] = z0 + j * 16 + iota16
                    pltpu.sync_copy(rowbufs[0], slab.at[idxbufs[0]])
                    if with_deg and _PD:
                        pltpu.sync_copy(zdbuf, slab_deg.at[idxbufs[0]])
            plsc.subcore_barrier()

            def group(g, _):
                edescs = []
                for b_ in range(NBUF):
                    off = e0 + (g * NBUF + b_) * CH
                    edescs.append(pltpu.async_copy(
                        edges_hbm.at[:, pl.ds(off, CH)], ebufs[b_],
                        esems[b_]))
                gdescs = []
                for b_ in range(NBUF):
                    edescs[b_].wait()
                    gdescs.append(pltpu.async_copy(
                        h_hbm.at[ebufs[b_].at[0]], rowbufs[b_], gsems[b_]))
                sdescs = []
                for b_ in range(NBUF):
                    compute_rows(ebufs[b_], idxbufs[b_], base, CH // 16)
                    gdescs[b_].wait()
                    sdescs.append(pltpu.async_copy(
                        rowbufs[b_], slab.at[idxbufs[b_]], ssems[b_],
                        add=True))
                    if with_deg:
                        sdescs.append(pltpu.async_copy(
                            onesb, slab_deg.at[idxbufs[b_]], dsems[b_],
                            add=True))
                for d_ in sdescs:
                    d_.wait()
                return 0

            if _PG:
                lax.fori_loop(0, NFULL // NBUF, group, 0)

            if _PT:
                toff = e0 + NFULL * CH
                pltpu.sync_copy(edges_hbm.at[:, pl.ds(toff, TAIL)],
                                ebufs[0].at[:, pl.ds(0, TAIL)])
                pltpu.async_copy(h_hbm.at[ebufs[0].at[0, pl.ds(0, TAIL)]],
                                 rowbufs[0].at[pl.ds(0, TAIL)],
                                 gsems[0]).wait()
                for j in range(TAIL // 16):
                    d_ = ebufs[0][1, pl.ds(j * 16, 16)]
                    t_ = ebufs[0][2, pl.ds(j * 16, 16)]
                    inb = (d_ >= base) & (d_ < base + NC)
                    row = jnp.where(inb, t_ * NC + (d_ - base),
                                    TRASH + (d_ & 127))
                    idxtail[pl.ds(j * 16, 16)] = row
                pltpu.sync_copy(rowbufs[0].at[pl.ds(0, TAIL)],
                                slab.at[idxtail], add=True)
                if with_deg:
                    pltpu.sync_copy(onesb.at[pl.ds(0, TAIL)],
                                    slab_deg.at[idxtail], add=True)

            plsc.subcore_barrier()
            if _PW:
                iota16 = lax.iota(jnp.int32, 16)

                def stage_out(starts):
                    for rr in range(REL):
                        for po in starts:
                            src0 = rr * NC + s * 400 + po
                            dst0 = base + s * 400 + po
                            for j in range(8):
                                idxbufs[0][pl.ds(j * 16, 16)] = (
                                    src0 + j * 16 + iota16)
                            pltpu.sync_copy(slab.at[idxbufs[0]], rowbufs[1])
                            pltpu.sync_copy(rowbufs[1],
                                            s_out.at[rr, pl.ds(dst0, CH)])
                            if with_deg:
                                pltpu.sync_copy(slab_deg.at[idxbufs[0]], zdbuf)
                                pltpu.sync_copy(
                                    zdbuf, deg_out.at[rr, pl.ds(dst0, CH)])

                stage_out([0, 128, 256, 272])
            plsc.subcore_barrier()

    return pl.kernel(body, out_type=tuple(out_type) if with_deg else out_type[0],
                     mesh=_mesh, scratch_types=scratch,
                     compiler_params=_sc_params)


_sc_agg_deg = _make_sc_agg(True)
_sc_agg_nodeg = _make_sc_agg(False)


@functools.partial(
    pl.kernel,
    out_type=(jax.ShapeDtypeStruct((2, 256, D), jnp.float32),
              jax.ShapeDtypeStruct((2, 256, 8), jnp.float32)),
    mesh=_mesh,
    scratch_types=[
        pltpu.VMEM((128, D), jnp.float32),
        pltpu.VMEM((128,), jnp.int32),
        pltpu.VMEM((128, 8), jnp.float32),
        pltpu.SemaphoreType.DMA,
        pltpu.VMEM_SHARED((256, D), jnp.float32),
        pltpu.VMEM_SHARED((256, 8), jnp.float32),
    ],
    compiler_params=_sc_params,
)
def _sc_pool(h_hbm, b_hbm, ones_hbm, zrow_hbm, zdeg_hbm, psum_out, pcnt_out,
             rows_v, bidx_v, onesb, sem, slab, slab_cnt):
    s = lax.axis_index("s")
    c = lax.axis_index("c")
    wid = s * 2 + c
    pltpu.sync_copy(ones_hbm.at[pl.ds(0, 128)], onesb)

    pltpu.sync_copy(zrow_hbm.at[pl.ds(0, 16)], slab.at[pl.ds(s * 16, 16)])
    pltpu.sync_copy(zdeg_hbm.at[pl.ds(0, 16)],
                    slab_cnt.at[pl.ds(s * 16, 16)])
    plsc.subcore_barrier()
    for k in range(13):
        ch = wid * 13 + k
        pltpu.sync_copy(h_hbm.at[pl.ds(ch * 128, 128)], rows_v)
        pltpu.sync_copy(b_hbm.at[pl.ds(ch * 128, 128)], bidx_v)
        pltpu.sync_copy(rows_v, slab.at[bidx_v], add=True)
        pltpu.sync_copy(onesb, slab_cnt.at[bidx_v], add=True)
    plsc.subcore_barrier()
    pltpu.sync_copy(slab.at[pl.ds(s * 16, 16)],
                    psum_out.at[c, pl.ds(s * 16, 16)])
    pltpu.sync_copy(slab_cnt.at[pl.ds(s * 16, 16)],
                    pcnt_out.at[c, pl.ds(s * 16, 16)])


def _dense_layer_body(h_ref, s_ref, deg_ref, w_ref, root_ref, b_ref, o_ref):
    acc = jnp.dot(h_ref[...], root_ref[...], preferred_element_type=jnp.float32)
    acc = acc + b_ref[...]
    for r in range(REL):
        invd = 1.0 / jnp.clip(deg_ref[r, :, 0], 1.0)
        acc = acc + jnp.dot(s_ref[r] * invd[:, None], w_ref[r],
                            preferred_element_type=jnp.float32)
    o_ref[...] = jnp.maximum(acc, 0.0)


def _dense_layer(h, s, deg, w, root, b):
    grid = NP // BLK
    return pl.pallas_call(
        _dense_layer_body,
        grid=(grid,),
        in_specs=[
            pl.BlockSpec((BLK, D), lambda i: (i, 0)),
            pl.BlockSpec((REL, BLK, D), lambda i: (0, i, 0)),
            pl.BlockSpec((REL, BLK, 8), lambda i: (0, i, 0)),
            pl.BlockSpec((REL, D, D), lambda i: (0, 0, 0)),
            pl.BlockSpec((D, D), lambda i: (0, 0)),
            pl.BlockSpec((1, D), lambda i: (0, 0)),
        ],
        out_specs=pl.BlockSpec((BLK, D), lambda i: (i, 0)),
        out_shape=jax.ShapeDtypeStruct((NP, D), jnp.float32),
    )(h, s, deg, w, root, b)


def _head_body(psum_ref, pcnt_ref, wl_ref, bl_ref, o_ref):
    sums = psum_ref[0, :B, :] + psum_ref[1, :B, :]
    cnt = pcnt_ref[0, :B, 0:1] + pcnt_ref[1, :B, 0:1]
    pooled = sums / jnp.clip(cnt, 1.0)
    o_ref[...] = jnp.dot(pooled, wl_ref[...],
                         preferred_element_type=jnp.float32) + bl_ref[...]


def _head(psum, pcnt, wl, bl):
    return pl.pallas_call(
        _head_body,
        out_shape=jax.ShapeDtypeStruct((B, CLS), jnp.float32),
    )(psum, pcnt, wl, bl)


def kernel(x, edge_index, edge_type, batch, emb, W1, root1, b1, W2, root2, b2, Wl, bl):
    emb0 = emb.at[0].set(0.0)
    x_pad = jnp.concatenate([x, jnp.zeros((NP - N,), jnp.int32)])
    b_pad = jnp.concatenate([batch, jnp.full((NP - N,), B, jnp.int32)])
    edges3 = jnp.concatenate([edge_index, edge_type[None]], axis=0)

    ones_c = jnp.ones((CH, 8), jnp.float32)
    zrow_c = jnp.zeros((400, D), jnp.float32)
    zdeg_c = jnp.zeros((CH, 8), jnp.float32)

    _STAGE = 3
    src, dst, ty = edges3[0], edges3[1], edges3[2]
    h0 = _sc_embed(x_pad, emb0)

    def _xla_agg(h):
        ids = ty * NP + dst
        s_ = jax.ops.segment_sum(h[src], ids, num_segments=REL * NP)
        d_ = jax.ops.segment_sum(jnp.ones((E,), jnp.float32), ids,
                                 num_segments=REL * NP)
        return s_.reshape(REL, NP, D), d_.reshape(REL, NP, 1)

    if _STAGE >= 2:
        s1, deg = _sc_agg_deg(h0, edges3, ones_c, zrow_c, zdeg_c)
    else:
        s1, deg = _xla_agg(h0)
    h1 = _dense_layer(h0, s1, deg, W1, root1, b1.reshape(1, D))
    if _STAGE >= 2:
        s2 = _sc_agg_nodeg(h1, edges3, ones_c, zrow_c, zdeg_c)
    else:
        s2, _deg2 = _xla_agg(h1)
    h2 = _dense_layer(h1, s2, deg, W2, root2, b2.reshape(1, D))
    if _STAGE >= 3:
        psum, pcnt = _sc_pool(h2, b_pad, ones_c, zrow_c, zdeg_c)
    else:
        hs = h2[:N]
        sums = jax.ops.segment_sum(hs, batch, num_segments=B)
        cnt = jax.ops.segment_sum(jnp.ones((N,), jnp.float32), batch,
                                  num_segments=B)
        psum = jnp.zeros((2, 256, D), jnp.float32).at[0, :B].set(sums)
        pcnt = jnp.zeros((2, 256, 1), jnp.float32).at[0, :B].set(cnt[:, None])
    return _head(psum, pcnt, Wl, bl.reshape(1, CLS))


# speedup vs baseline: 3.8196x; 1.0574x over previous
import functools
import jax
import jax.numpy as jnp
from jax import lax
from jax.experimental import pallas as pl
from jax.experimental.pallas import tpu as pltpu
from jax.experimental.pallas import tpu_sc as plsc

N = 50000
E = 800000
REL = 3
D = 64
B = 128
CLS = 2

BLK = 256
NP = 53248

NC = 6400
NRANGE_PER_CORE = 4
TRASH = REL * NC
SLAB = 19328
CH = 128
NBUF = 3
EPT = 50000
NFULL = 390
_PZ = True
_PT = True
_PW = True
_PD = True
_PG = True
TAIL = 80

_mesh = plsc.VectorSubcoreMesh(core_axis_name="c", subcore_axis_name="s")
_sc_params = pltpu.CompilerParams(use_tc_tiling_on_sc=False)


@functools.partial(
    pl.kernel,
    out_type=jax.ShapeDtypeStruct((NP, D), jnp.float32),
    mesh=_mesh,
    scratch_types=[
        pltpu.VMEM((1664,), jnp.int32),
        pltpu.VMEM((1664, D), jnp.float32),
        pltpu.SemaphoreType.DMA,
    ],
    compiler_params=_sc_params,
)
def _sc_embed(x_hbm, emb_hbm, out_hbm, idx_v, rows_v, sem):
    s = lax.axis_index("s")
    c = lax.axis_index("c")
    wid = s * 2 + c
    base = wid * 1664
    pltpu.sync_copy(x_hbm.at[pl.ds(base, 1664)], idx_v)
    descs = []
    for k in range(13):
        descs.append(pltpu.async_copy(
            emb_hbm.at[idx_v.at[pl.ds(k * 128, 128)]],
            rows_v.at[pl.ds(k * 128, 128)], sem))
    for d_ in descs:
        d_.wait()
    pltpu.sync_copy(rows_v, out_hbm.at[pl.ds(base, 1664)])


def _make_sc_agg(with_deg: bool, nbuf: int = NBUF):
    out_type = [jax.ShapeDtypeStruct((REL, NP, D), jnp.float32)]
    if with_deg:
        out_type.append(jax.ShapeDtypeStruct((REL, NP, 8), jnp.float32))
    scratch = [
        [pltpu.VMEM((3, CH), jnp.int32) for _ in range(nbuf)],
        [pltpu.VMEM((CH, D), jnp.float32) for _ in range(nbuf)],
        [pltpu.VMEM((CH,), jnp.int32) for _ in range(nbuf)],
        pltpu.VMEM((TAIL,), jnp.int32),
        pltpu.VMEM((CH, 8), jnp.float32),
        pltpu.VMEM((CH, 8), jnp.float32),
        [pltpu.SemaphoreType.DMA for _ in range(nbuf)],
        [pltpu.SemaphoreType.DMA for _ in range(nbuf)],
        [pltpu.SemaphoreType.DMA for _ in range(nbuf)],
        [pltpu.SemaphoreType.DMA for _ in range(nbuf)],
        pltpu.VMEM_SHARED((SLAB, D), jnp.float32),
        pltpu.VMEM_SHARED((SLAB, 8) if with_deg else (CH, 8), jnp.float32),
    ]

    def compute_rows(ebuf, idxbuf, base, n16):
        for j in range(n16):
            d_ = ebuf[1, pl.ds(j * 16, 16)]
            t_ = ebuf[2, pl.ds(j * 16, 16)]
            inb = (d_ >= base) & (d_ < base + NC)
            row = jnp.where(inb, t_ * NC + (d_ - base), TRASH + (d_ & 127))
            idxbuf[pl.ds(j * 16, 16)] = row

    def body(h_hbm, edges_hbm, ones_hbm, zrow_hbm, zdeg_hbm, *rest):
        if with_deg:
            s_out, deg_out = rest[0], rest[1]
            rest = rest[2:]
        else:
            s_out = rest[0]
            rest = rest[1:]
        (ebufs, rowbufs, idxbufs, idxtail, onesb, zdbuf, esems, gsems,
         ssems, dsems, slab, slab_deg) = rest

        s = lax.axis_index("s")
        c = lax.axis_index("c")
        pltpu.sync_copy(ones_hbm, onesb)
        e0 = s * EPT

        for p in range(NRANGE_PER_CORE):
            r = c * NRANGE_PER_CORE + p
            base = r * NC
            if _PZ:
                pltpu.sync_copy(zrow_hbm.at[pl.ds(0, CH)], rowbufs[0])
                pltpu.sync_copy(zdeg_hbm.at[pl.ds(0, CH)], zdbuf)
                iota16 = lax.iota(jnp.int32, 16)
                for z in range(10):
                    z0 = s * 1208 + (z * 128 if z < 9 else 1080)
                    for j in range(8):
                        idxbufs[0][pl.ds(j * 16, 16)] = z0 + j * 16 + iota16
                    pltpu.sync_copy(rowbufs[0], slab.at[idxbufs[0]])
                    if with_deg and _PD:
                        pltpu.sync_copy(zdbuf, slab_deg.at[idxbufs[0]])
            plsc.subcore_barrier()

            def group(g, _):
                edescs = []
                for b_ in range(nbuf):
                    off = e0 + (g * nbuf + b_) * CH
                    edescs.append(pltpu.async_copy(
                        edges_hbm.at[:, pl.ds(off, CH)], ebufs[b_],
                        esems[b_]))
                gdescs = []
                for b_ in range(nbuf):
                    edescs[b_].wait()
                    gdescs.append(pltpu.async_copy(
                        h_hbm.at[ebufs[b_].at[0]], rowbufs[b_], gsems[b_]))
                sdescs = []
                for b_ in range(nbuf):
                    compute_rows(ebufs[b_], idxbufs[b_], base, CH // 16)
                    gdescs[b_].wait()
                    sdescs.append(pltpu.async_copy(
                        rowbufs[b_], slab.at[idxbufs[b_]], ssems[b_],
                        add=True))
                    if with_deg:
                        sdescs.append(pltpu.async_copy(
                            onesb, slab_deg.at[idxbufs[b_]], dsems[b_],
                            add=True))
                for d_ in sdescs:
                    d_.wait()
                return 0

            if _PG:
                lax.fori_loop(0, NFULL // nbuf, group, 0)

            if _PT:
                toff = e0 + NFULL * CH
                pltpu.sync_copy(edges_hbm.at[:, pl.ds(toff, TAIL)],
                                ebufs[0].at[:, pl.ds(0, TAIL)])
                pltpu.async_copy(h_hbm.at[ebufs[0].at[0, pl.ds(0, TAIL)]],
                                 rowbufs[0].at[pl.ds(0, TAIL)],
                                 gsems[0]).wait()
                for j in range(TAIL // 16):
                    d_ = ebufs[0][1, pl.ds(j * 16, 16)]
                    t_ = ebufs[0][2, pl.ds(j * 16, 16)]
                    inb = (d_ >= base) & (d_ < base + NC)
                    row = jnp.where(inb, t_ * NC + (d_ - base),
                                    TRASH + (d_ & 127))
                    idxtail[pl.ds(j * 16, 16)] = row
                pltpu.sync_copy(rowbufs[0].at[pl.ds(0, TAIL)],
                                slab.at[idxtail], add=True)
                if with_deg:
                    pltpu.sync_copy(onesb.at[pl.ds(0, TAIL)],
                                    slab_deg.at[idxtail], add=True)

            plsc.subcore_barrier()
            if _PW:
                iota16 = lax.iota(jnp.int32, 16)

                def stage_out(starts):
                    for rr in range(REL):
                        for po in starts:
                            src0 = rr * NC + s * 400 + po
                            dst0 = base + s * 400 + po
                            for j in range(8):
                                idxbufs[0][pl.ds(j * 16, 16)] = (
                                    src0 + j * 16 + iota16)
                            pltpu.sync_copy(slab.at[idxbufs[0]], rowbufs[1])
                            pltpu.sync_copy(rowbufs[1],
                                            s_out.at[rr, pl.ds(dst0, CH)])
                            if with_deg:
                                pltpu.sync_copy(slab_deg.at[idxbufs[0]], zdbuf)
                                pltpu.sync_copy(
                                    zdbuf, deg_out.at[rr, pl.ds(dst0, CH)])

                stage_out([0, 128, 256, 272])
            plsc.subcore_barrier()

    return pl.kernel(body, out_type=tuple(out_type) if with_deg else out_type[0],
                     mesh=_mesh, scratch_types=scratch,
                     compiler_params=_sc_params)


_sc_agg_deg = _make_sc_agg(True)
_sc_agg_nodeg = _make_sc_agg(False, nbuf=5)


@functools.partial(
    pl.kernel,
    out_type=(jax.ShapeDtypeStruct((2, 256, D), jnp.float32),
              jax.ShapeDtypeStruct((2, 256, 8), jnp.float32)),
    mesh=_mesh,
    scratch_types=[
        pltpu.VMEM((128, D), jnp.float32),
        pltpu.VMEM((128,), jnp.int32),
        pltpu.VMEM((128, 8), jnp.float32),
        pltpu.SemaphoreType.DMA,
        pltpu.VMEM_SHARED((256, D), jnp.float32),
        pltpu.VMEM_SHARED((256, 8), jnp.float32),
    ],
    compiler_params=_sc_params,
)
def _sc_pool(h_hbm, b_hbm, ones_hbm, zrow_hbm, zdeg_hbm, psum_out, pcnt_out,
             rows_v, bidx_v, onesb, sem, slab, slab_cnt):
    s = lax.axis_index("s")
    c = lax.axis_index("c")
    wid = s * 2 + c
    pltpu.sync_copy(ones_hbm.at[pl.ds(0, 128)], onesb)

    pltpu.sync_copy(zrow_hbm.at[pl.ds(0, 16)], slab.at[pl.ds(s * 16, 16)])
    pltpu.sync_copy(zdeg_hbm.at[pl.ds(0, 16)],
                    slab_cnt.at[pl.ds(s * 16, 16)])
    plsc.subcore_barrier()
    for k in range(13):
        ch = wid * 13 + k
        pltpu.sync_copy(h_hbm.at[pl.ds(ch * 128, 128)], rows_v)
        pltpu.sync_copy(b_hbm.at[pl.ds(ch * 128, 128)], bidx_v)
        pltpu.sync_copy(rows_v, slab.at[bidx_v], add=True)
        pltpu.sync_copy(onesb, slab_cnt.at[bidx_v], add=True)
    plsc.subcore_barrier()
    pltpu.sync_copy(slab.at[pl.ds(s * 16, 16)],
                    psum_out.at[c, pl.ds(s * 16, 16)])
    pltpu.sync_copy(slab_cnt.at[pl.ds(s * 16, 16)],
                    pcnt_out.at[c, pl.ds(s * 16, 16)])


def _dense_layer_body(h_ref, s_ref, deg_ref, w_ref, root_ref, b_ref, o_ref):
    acc = jnp.dot(h_ref[...], root_ref[...], preferred_element_type=jnp.float32)
    acc = acc + b_ref[...]
    for r in range(REL):
        invd = 1.0 / jnp.clip(deg_ref[r, :, 0], 1.0)
        acc = acc + jnp.dot(s_ref[r] * invd[:, None], w_ref[r],
                            preferred_element_type=jnp.float32)
    o_ref[...] = jnp.maximum(acc, 0.0)


def _dense_layer(h, s, deg, w, root, b):
    grid = NP // BLK
    return pl.pallas_call(
        _dense_layer_body,
        grid=(grid,),
        in_specs=[
            pl.BlockSpec((BLK, D), lambda i: (i, 0)),
            pl.BlockSpec((REL, BLK, D), lambda i: (0, i, 0)),
            pl.BlockSpec((REL, BLK, 8), lambda i: (0, i, 0)),
            pl.BlockSpec((REL, D, D), lambda i: (0, 0, 0)),
            pl.BlockSpec((D, D), lambda i: (0, 0)),
            pl.BlockSpec((1, D), lambda i: (0, 0)),
        ],
        out_specs=pl.BlockSpec((BLK, D), lambda i: (i, 0)),
        out_shape=jax.ShapeDtypeStruct((NP, D), jnp.float32),
    )(h, s, deg, w, root, b)


def _head_body(psum_ref, pcnt_ref, wl_ref, bl_ref, o_ref):
    sums = psum_ref[0, :B, :] + psum_ref[1, :B, :]
    cnt = pcnt_ref[0, :B, 0:1] + pcnt_ref[1, :B, 0:1]
    pooled = sums / jnp.clip(cnt, 1.0)
    o_ref[...] = jnp.dot(pooled, wl_ref[...],
                         preferred_element_type=jnp.float32) + bl_ref[...]


def _head(psum, pcnt, wl, bl):
    return pl.pallas_call(
        _head_body,
        out_shape=jax.ShapeDtypeStruct((B, CLS), jnp.float32),
    )(psum, pcnt, wl, bl)


def kernel(x, edge_index, edge_type, batch, emb, W1, root1, b1, W2, root2, b2, Wl, bl):
    emb0 = emb.at[0].set(0.0)
    x_pad = jnp.concatenate([x, jnp.zeros((NP - N,), jnp.int32)])
    b_pad = jnp.concatenate([batch, jnp.full((NP - N,), B, jnp.int32)])
    edges3 = jnp.concatenate([edge_index, edge_type[None]], axis=0)

    ones_c = jnp.ones((CH, 8), jnp.float32)
    zrow_c = jnp.zeros((400, D), jnp.float32)
    zdeg_c = jnp.zeros((CH, 8), jnp.float32)

    _STAGE = 3
    src, dst, ty = edges3[0], edges3[1], edges3[2]
    h0 = _sc_embed(x_pad, emb0)

    def _xla_agg(h):
        ids = ty * NP + dst
        s_ = jax.ops.segment_sum(h[src], ids, num_segments=REL * NP)
        d_ = jax.ops.segment_sum(jnp.ones((E,), jnp.float32), ids,
                                 num_segments=REL * NP)
        return s_.reshape(REL, NP, D), d_.reshape(REL, NP, 1)

    if _STAGE >= 2:
        s1, deg = _sc_agg_deg(h0, edges3, ones_c, zrow_c, zdeg_c)
    else:
        s1, deg = _xla_agg(h0)
    h1 = _dense_layer(h0, s1, deg, W1, root1, b1.reshape(1, D))
    if _STAGE >= 2:
        s2 = _sc_agg_nodeg(h1, edges3, ones_c, zrow_c, zdeg_c)
    else:
        s2, _deg2 = _xla_agg(h1)
    h2 = _dense_layer(h1, s2, deg, W2, root2, b2.reshape(1, D))
    if _STAGE >= 3:
        psum, pcnt = _sc_pool(h2, b_pad, ones_c, zrow_c, zdeg_c)
    else:
        hs = h2[:N]
        sums = jax.ops.segment_sum(hs, batch, num_segments=B)
        cnt = jax.ops.segment_sum(jnp.ones((N,), jnp.float32), batch,
                                  num_segments=B)
        psum = jnp.zeros((2, 256, D), jnp.float32).at[0, :B].set(sums)
        pcnt = jnp.zeros((2, 256, 1), jnp.float32).at[0, :B].set(cnt[:, None])
    return _head(psum, pcnt, Wl, bl.reshape(1, CLS))


# speedup vs baseline: 3.8218x; 1.0006x over previous
import functools
import jax
import jax.numpy as jnp
from jax import lax
from jax.experimental import pallas as pl
from jax.experimental.pallas import tpu as pltpu
from jax.experimental.pallas import tpu_sc as plsc

N = 50000
E = 800000
REL = 3
D = 64
B = 128
CLS = 2

BLK = 256
NP = 53248

NC = 6400
NRANGE_PER_CORE = 4
TRASH = REL * NC
SLAB = 19328
CH = 128
NBUF = 3
EPT = 50000
NFULL = 390
TAIL = 80

_mesh = plsc.VectorSubcoreMesh(core_axis_name="c", subcore_axis_name="s")
_sc_params = pltpu.CompilerParams(use_tc_tiling_on_sc=False)


@functools.partial(
    pl.kernel,
    out_type=jax.ShapeDtypeStruct((NP, D), jnp.float32),
    mesh=_mesh,
    scratch_types=[
        pltpu.VMEM((1664,), jnp.int32),
        pltpu.VMEM((1664, D), jnp.float32),
        pltpu.SemaphoreType.DMA,
    ],
    compiler_params=_sc_params,
)
def _sc_embed(x_hbm, emb_hbm, out_hbm, idx_v, rows_v, sem):
    s = lax.axis_index("s")
    c = lax.axis_index("c")
    wid = s * 2 + c
    base = wid * 1664
    pltpu.sync_copy(x_hbm.at[pl.ds(base, 1664)], idx_v)
    descs = []
    for k in range(13):
        descs.append(pltpu.async_copy(
            emb_hbm.at[idx_v.at[pl.ds(k * 128, 128)]],
            rows_v.at[pl.ds(k * 128, 128)], sem))
    for d_ in descs:
        d_.wait()
    pltpu.sync_copy(rows_v, out_hbm.at[pl.ds(base, 1664)])


def _make_sc_agg(with_deg: bool, nbuf: int = NBUF):
    out_type = [jax.ShapeDtypeStruct((REL, NP, D), jnp.float32)]
    if with_deg:
        out_type.append(jax.ShapeDtypeStruct((REL, NP, 8), jnp.float32))
    scratch = [
        [pltpu.VMEM((3, CH), jnp.int32) for _ in range(nbuf)],
        [pltpu.VMEM((CH, D), jnp.float32) for _ in range(nbuf)],
        [pltpu.VMEM((CH,), jnp.int32) for _ in range(nbuf)],
        pltpu.VMEM((TAIL,), jnp.int32),
        pltpu.VMEM((CH, 8), jnp.float32),
        pltpu.VMEM((CH, 8), jnp.float32),
        [pltpu.SemaphoreType.DMA for _ in range(nbuf)],
        [pltpu.SemaphoreType.DMA for _ in range(nbuf)],
        [pltpu.SemaphoreType.DMA for _ in range(nbuf)],
        [pltpu.SemaphoreType.DMA for _ in range(nbuf)],
        pltpu.VMEM_SHARED((SLAB, D), jnp.float32),
        pltpu.VMEM_SHARED((SLAB, 8) if with_deg else (CH, 8), jnp.float32),
    ]

    def compute_rows(ebuf, idxbuf, base, n16):
        for j in range(n16):
            d_ = ebuf[1, pl.ds(j * 16, 16)]
            t_ = ebuf[2, pl.ds(j * 16, 16)]
            inb = (d_ >= base) & (d_ < base + NC)
            row = jnp.where(inb, t_ * NC + (d_ - base), TRASH + (d_ & 127))
            idxbuf[pl.ds(j * 16, 16)] = row

    def body(h_hbm, edges_hbm, ones_hbm, zrow_hbm, zdeg_hbm, *rest):
        if with_deg:
            s_out, deg_out = rest[0], rest[1]
            rest = rest[2:]
        else:
            s_out = rest[0]
            rest = rest[1:]
        (ebufs, rowbufs, idxbufs, idxtail, onesb, zdbuf, esems, gsems,
         ssems, dsems, slab, slab_deg) = rest

        s = lax.axis_index("s")
        c = lax.axis_index("c")
        pltpu.sync_copy(ones_hbm, onesb)
        e0 = s * EPT

        for p in range(NRANGE_PER_CORE):
            r = c * NRANGE_PER_CORE + p
            base = r * NC
            pltpu.sync_copy(zrow_hbm.at[pl.ds(0, CH)], rowbufs[0])
            pltpu.sync_copy(zdeg_hbm.at[pl.ds(0, CH)], zdbuf)
            iota16 = lax.iota(jnp.int32, 16)
            for z in range(10):
                z0 = s * 1208 + (z * 128 if z < 9 else 1080)
                for j in range(8):
                    idxbufs[0][pl.ds(j * 16, 16)] = z0 + j * 16 + iota16
                pltpu.sync_copy(rowbufs[0], slab.at[idxbufs[0]])
                if with_deg:
                    pltpu.sync_copy(zdbuf, slab_deg.at[idxbufs[0]])
            plsc.subcore_barrier()

            def group(g, _):
                edescs = []
                for b_ in range(nbuf):
                    off = e0 + (g * nbuf + b_) * CH
                    edescs.append(pltpu.async_copy(
                        edges_hbm.at[:, pl.ds(off, CH)], ebufs[b_],
                        esems[b_]))
                gdescs = []
                for b_ in range(nbuf):
                    edescs[b_].wait()
                    gdescs.append(pltpu.async_copy(
                        h_hbm.at[ebufs[b_].at[0]], rowbufs[b_], gsems[b_]))
                sdescs = []
                for b_ in range(nbuf):
                    compute_rows(ebufs[b_], idxbufs[b_], base, CH // 16)
                    gdescs[b_].wait()
                    sdescs.append(pltpu.async_copy(
                        rowbufs[b_], slab.at[idxbufs[b_]], ssems[b_],
                        add=True))
                    if with_deg:
                        sdescs.append(pltpu.async_copy(
                            onesb, slab_deg.at[idxbufs[b_]], dsems[b_],
                            add=True))
                for d_ in sdescs:
                    d_.wait()
                return 0

            lax.fori_loop(0, NFULL // nbuf, group, 0)

            toff = e0 + NFULL * CH
            pltpu.sync_copy(edges_hbm.at[:, pl.ds(toff, TAIL)],
                            ebufs[0].at[:, pl.ds(0, TAIL)])
            pltpu.async_copy(h_hbm.at[ebufs[0].at[0, pl.ds(0, TAIL)]],
                             rowbufs[0].at[pl.ds(0, TAIL)],
                             gsems[0]).wait()
            for j in range(TAIL // 16):
                d_ = ebufs[0][1, pl.ds(j * 16, 16)]
                t_ = ebufs[0][2, pl.ds(j * 16, 16)]
                inb = (d_ >= base) & (d_ < base + NC)
                row = jnp.where(inb, t_ * NC + (d_ - base),
                                TRASH + (d_ & 127))
                idxtail[pl.ds(j * 16, 16)] = row
            pltpu.sync_copy(rowbufs[0].at[pl.ds(0, TAIL)],
                            slab.at[idxtail], add=True)
            if with_deg:
                pltpu.sync_copy(onesb.at[pl.ds(0, TAIL)],
                                slab_deg.at[idxtail], add=True)

            plsc.subcore_barrier()
            for rr in range(REL):
                for po in (0, 128, 256, 272):
                    src0 = rr * NC + s * 400 + po
                    dst0 = base + s * 400 + po
                    for j in range(8):
                        idxbufs[0][pl.ds(j * 16, 16)] = src0 + j * 16 + iota16
                    pltpu.sync_copy(slab.at[idxbufs[0]], rowbufs[1])
                    pltpu.sync_copy(rowbufs[1],
                                    s_out.at[rr, pl.ds(dst0, CH)])
                    if with_deg:
                        pltpu.sync_copy(slab_deg.at[idxbufs[0]], zdbuf)
                        pltpu.sync_copy(zdbuf,
                                        deg_out.at[rr, pl.ds(dst0, CH)])
            plsc.subcore_barrier()

    return pl.kernel(body, out_type=tuple(out_type) if with_deg else out_type[0],
                     mesh=_mesh, scratch_types=scratch,
                     compiler_params=_sc_params)


_sc_agg_deg = _make_sc_agg(True)
_sc_agg_nodeg = _make_sc_agg(False, nbuf=5)


@functools.partial(
    pl.kernel,
    out_type=(jax.ShapeDtypeStruct((2, 256, D), jnp.float32),
              jax.ShapeDtypeStruct((2, 256, 8), jnp.float32)),
    mesh=_mesh,
    scratch_types=[
        pltpu.VMEM((128, D), jnp.float32),
        pltpu.VMEM((128,), jnp.int32),
        pltpu.VMEM((128, 8), jnp.float32),
        pltpu.SemaphoreType.DMA,
        pltpu.VMEM_SHARED((256, D), jnp.float32),
        pltpu.VMEM_SHARED((256, 8), jnp.float32),
    ],
    compiler_params=_sc_params,
)
def _sc_pool(h_hbm, b_hbm, ones_hbm, zrow_hbm, zdeg_hbm, psum_out, pcnt_out,
             rows_v, bidx_v, onesb, sem, slab, slab_cnt):
    s = lax.axis_index("s")
    c = lax.axis_index("c")
    wid = s * 2 + c
    pltpu.sync_copy(ones_hbm.at[pl.ds(0, 128)], onesb)

    pltpu.sync_copy(zrow_hbm.at[pl.ds(0, 16)], slab.at[pl.ds(s * 16, 16)])
    pltpu.sync_copy(zdeg_hbm.at[pl.ds(0, 16)],
                    slab_cnt.at[pl.ds(s * 16, 16)])
    plsc.subcore_barrier()
    for k in range(13):
        ch = wid * 13 + k
        pltpu.sync_copy(h_hbm.at[pl.ds(ch * 128, 128)], rows_v)
        pltpu.sync_copy(b_hbm.at[pl.ds(ch * 128, 128)], bidx_v)
        pltpu.sync_copy(rows_v, slab.at[bidx_v], add=True)
        pltpu.sync_copy(onesb, slab_cnt.at[bidx_v], add=True)
    plsc.subcore_barrier()
    pltpu.sync_copy(slab.at[pl.ds(s * 16, 16)],
                    psum_out.at[c, pl.ds(s * 16, 16)])
    pltpu.sync_copy(slab_cnt.at[pl.ds(s * 16, 16)],
                    pcnt_out.at[c, pl.ds(s * 16, 16)])


def _dense_layer_body(h_ref, s_ref, deg_ref, w_ref, root_ref, b_ref, o_ref):
    acc = jnp.dot(h_ref[...], root_ref[...], preferred_element_type=jnp.float32)
    acc = acc + b_ref[...]
    for r in range(REL):
        invd = 1.0 / jnp.clip(deg_ref[r, :, 0], 1.0)
        acc = acc + jnp.dot(s_ref[r] * invd[:, None], w_ref[r],
                            preferred_element_type=jnp.float32)
    o_ref[...] = jnp.maximum(acc, 0.0)


def _dense_layer(h, s, deg, w, root, b):
    grid = NP // BLK
    return pl.pallas_call(
        _dense_layer_body,
        grid=(grid,),
        in_specs=[
            pl.BlockSpec((BLK, D), lambda i: (i, 0)),
            pl.BlockSpec((REL, BLK, D), lambda i: (0, i, 0)),
            pl.BlockSpec((REL, BLK, 8), lambda i: (0, i, 0)),
            pl.BlockSpec((REL, D, D), lambda i: (0, 0, 0)),
            pl.BlockSpec((D, D), lambda i: (0, 0)),
            pl.BlockSpec((1, D), lambda i: (0, 0)),
        ],
        out_specs=pl.BlockSpec((BLK, D), lambda i: (i, 0)),
        out_shape=jax.ShapeDtypeStruct((NP, D), jnp.float32),
    )(h, s, deg, w, root, b)


def _head_body(psum_ref, pcnt_ref, wl_ref, bl_ref, o_ref):
    sums = psum_ref[0, :B, :] + psum_ref[1, :B, :]
    cnt = pcnt_ref[0, :B, 0:1] + pcnt_ref[1, :B, 0:1]
    pooled = sums / jnp.clip(cnt, 1.0)
    o_ref[...] = jnp.dot(pooled, wl_ref[...],
                         preferred_element_type=jnp.float32) + bl_ref[...]


def _head(psum, pcnt, wl, bl):
    return pl.pallas_call(
        _head_body,
        out_shape=jax.ShapeDtypeStruct((B, CLS), jnp.float32),
    )(psum, pcnt, wl, bl)


def kernel(x, edge_index, edge_type, batch, emb, W1, root1, b1, W2, root2, b2, Wl, bl):
    emb0 = emb.at[0].set(0.0)
    x_pad = jnp.concatenate([x, jnp.zeros((NP - N,), jnp.int32)])
    b_pad = jnp.concatenate([batch, jnp.full((NP - N,), B, jnp.int32)])
    edges3 = jnp.concatenate([edge_index, edge_type[None]], axis=0)

    ones_c = jnp.ones((CH, 8), jnp.float32)
    zrow_c = jnp.zeros((400, D), jnp.float32)
    zdeg_c = jnp.zeros((CH, 8), jnp.float32)

    h0 = _sc_embed(x_pad, emb0)
    s1, deg = _sc_agg_deg(h0, edges3, ones_c, zrow_c, zdeg_c)
    h1 = _dense_layer(h0, s1, deg, W1, root1, b1.reshape(1, D))
    s2 = _sc_agg_nodeg(h1, edges3, ones_c, zrow_c, zdeg_c)
    h2 = _dense_layer(h1, s2, deg, W2, root2, b2.reshape(1, D))
    psum, pcnt = _sc_pool(h2, b_pad, ones_c, zrow_c, zdeg_c)
    return _head(psum, pcnt, Wl, bl.reshape(1, CLS))
